# baseline (speedup 1.0000x reference)
import jax
import jax.numpy as jnp
from jax import lax
from jax.experimental import pallas as pl
from jax.experimental.pallas import tpu as pltpu

N_T = 256
C = 128
EPS = 1e-6
B = 4
D = 1024
TD = 256
JD = D // TD
N_S = 16


def _block_sums(tu, noise, x_row, beta_eff, tbl_small, const_row, W, T_row,
                lkd, lko):
    cond = tu < beta_eff
    S_i = jnp.sum(cond, axis=1, keepdims=True)
    S_col = S_i.astype(jnp.float32)

    i16 = lax.broadcasted_iota(jnp.int32, (TD, N_S), 1)
    ohS = (i16 == S_i).astype(jnp.float32)
    ohS1 = (i16 == (S_i - 1)).astype(jnp.float32)
    vS = jnp.dot(ohS, tbl_small, preferred_element_type=jnp.float32,
                 precision=lax.Precision.HIGHEST)
    vS1 = jnp.dot(ohS1, tbl_small[:, 0:1],
                  preferred_element_type=jnp.float32,
                  precision=lax.Precision.HIGHEST)
    vS = jnp.where(S_i < N_S, vS, const_row)
    dS1 = jnp.where((S_i >= 1) & (S_i < N_S + 1), vS1, const_row[:, 0:1])
    dS = vS[:, 0:1]
    oS = vS[:, 1:2]
    oS1 = dS

    ohxT = (lax.broadcasted_iota(jnp.int32, (C, TD), 0) == x_row)
    c_col = lax.broadcasted_iota(jnp.int32, (C, 1), 0).astype(jnp.float32)
    x_col = lax.dot_general(ohxT.astype(jnp.float32), c_col,
                            (((0,), (0,)), ((), ())),
                            preferred_element_type=jnp.float32)
    x_i = x_col.astype(jnp.int32)
    c_iota = lax.broadcasted_iota(jnp.int32, (TD, C), 1)
    bx = c_iota == x_i

    la = jnp.log(dS + EPS)
    lb = jnp.log(oS + EPS)
    logits = jnp.where(bx, la, lb)
    nz = jnp.clip(noise, EPS, 1.0)
    g = -jnp.log(-jnp.log(nz))
    y = logits + g
    m = jnp.max(y, axis=1, keepdims=True)
    x_t = jnp.min(jnp.where(y == m, c_iota, C), axis=1, keepdims=True)
    btm = c_iota == x_t
    OH_t = btm.astype(jnp.float32)

    pred_x0 = jnp.dot(OH_t, W,
                      preferred_element_type=jnp.float32,
                      precision=lax.Precision.HIGHEST) + T_row

    mx = jnp.max(pred_x0, axis=1, keepdims=True)
    sh = pred_x0 - mx
    e = jnp.exp(sh)
    se = jnp.sum(e, axis=1, keepdims=True)
    lse = jnp.log(se)

    lp_tok = jnp.sum(jnp.where(bx, sh, 0.0), axis=1, keepdims=True) - lse
    ce_part = jnp.sum(lp_tok)

    lf1 = jnp.where(btm, lkd, lko)

    denom = 1.0 + (C - 1) * EPS
    f2d = (dS1 + EPS * (1.0 - dS1)) / denom
    f2o = (oS1 + EPS * (1.0 - oS1)) / denom
    true_q = lf1 + jnp.where(bx, jnp.log(f2d + EPS), jnp.log(f2o + EPS))

    fact2_p = oS1 + ((dS1 - oS1) / se) * e
    pred_q = lf1 + jnp.log(fact2_p + EPS)

    mt = jnp.max(true_q, axis=1, keepdims=True)
    sht = true_q - mt
    et = jnp.exp(sht)
    st = jnp.sum(et, axis=1, keepdims=True)
    mp = jnp.max(pred_q, axis=1, keepdims=True)
    shp = pred_q - mp
    ep = jnp.exp(shp)
    sp = jnp.sum(ep, axis=1, keepdims=True)
    dot_ts = jnp.sum(et * (sht - shp), axis=1, keepdims=True)
    kls = dot_ts / st + (jnp.log(sp) - jnp.log(st))

    vb_raw = jnp.sum(kls * S_col)
    return vb_raw, ce_part


def _d3pm_kernel(t_ref, tu_ref, noise_ref, x_ref, beta_ref, tbl_ref, W_ref,
                 T_ref, K_ref, out_ref, acc_ref):
    b = pl.program_id(0)
    tb = t_ref[0, b]

    beta_row = beta_ref[...]
    s1 = lax.broadcasted_iota(jnp.int32, (1, N_T), 1)
    beta_eff = jnp.where(s1 < tb, beta_row, 0.0)
    tbl_small = tbl_ref[0:N_S, 0, 0:2]
    const_row = tbl_ref[N_S:N_S + 1, 0, 0:2]
    W = W_ref[...]
    T_row = T_ref[pl.ds(tb, 1), :]
    lkd = jnp.log(K_ref[0, 0] + EPS)
    lko = jnp.log(K_ref[0, 1] + EPS)

    def _body(h, carry):
        vb_acc, ce_acc = carry
        lo = h * TD
        vb_h, ce_h = _block_sums(tu_ref[pl.ds(lo, TD), :],
                                 noise_ref[pl.ds(lo, TD), :],
                                 x_ref[:, pl.ds(lo, TD)], beta_eff, tbl_small,
                                 const_row, W, T_row, lkd, lko)
        return vb_acc + vb_h, ce_acc + ce_h

    vb_raw, ce_part = lax.fori_loop(0, D // TD, _body, (0.0, 0.0))

    bt_val = jnp.sum(jnp.where(s1 == tb, beta_row, 0.0))
    cum_val = jnp.sum(jnp.where(s1 <= tb, beta_row, 0.0))
    w_b = bt_val / cum_val
    vb_part = vb_raw * (w_b * float(N_T))

    @pl.when(b == 0)
    def _init():
        acc_ref[0, 0] = 0.0
        acc_ref[0, 1] = 0.0

    acc_ref[0, 0] += vb_part
    acc_ref[0, 1] += ce_part

    @pl.when(b == B - 1)
    def _fin():
        n_tok = float(B * D)
        total = 0.001 * (-acc_ref[0, 1] / n_tok) + acc_ref[0, 0] / n_tok
        out_ref[...] = jnp.broadcast_to(total, (1, 1))


def kernel(x, t, trans_u, noise, W, T_emb, K, K_powers, beta_t):
    tu2 = trans_u.reshape(D, B * N_T)
    x3 = x.reshape(B, 1, D).astype(jnp.int32)
    t2 = t.reshape(1, B).astype(jnp.int32)
    beta2 = beta_t.reshape(1, N_T)

    out = pl.pallas_call(
        _d3pm_kernel,
        grid=(B,),
        in_specs=[
            pl.BlockSpec(memory_space=pltpu.SMEM),
            pl.BlockSpec((D, N_T), lambda b: (0, b)),
            pl.BlockSpec((None, D, C), lambda b: (b, 0, 0)),
            pl.BlockSpec((None, 1, D), lambda b: (b, 0, 0)),
            pl.BlockSpec((1, N_T), lambda b: (0, 0)),
            pl.BlockSpec((N_S + 1, 8, C), lambda b: (0, 0, 0)),
            pl.BlockSpec((C, C), lambda b: (0, 0)),
            pl.BlockSpec((N_T, C), lambda b: (0, 0)),
            pl.BlockSpec((C, C), lambda b: (0, 0)),
        ],
        out_specs=pl.BlockSpec((1, 1), lambda b: (0, 0)),
        out_shape=jax.ShapeDtypeStruct((1, 1), jnp.float32),
        scratch_shapes=[pltpu.SMEM((1, 2), jnp.float32)],
    )(t2, tu2, noise, x3, beta2, K_powers, W, T_emb, K)
    return out[0, 0]


# speedup vs baseline: 1.2965x; 1.2965x over previous
import jax
import jax.numpy as jnp
from jax import lax
from jax.experimental import pallas as pl
from jax.experimental.pallas import tpu as pltpu

N_T = 256
C = 128
EPS = 1e-6
B = 4
D = 1024
TD = 512
JD = D // TD
N_S = 16


def _block_sums(tu, noise, x_row, beta_eff, tbl_small, const_row, W, T_row,
                lkd, lko):
    cond = tu < beta_eff
    S_i = jnp.sum(cond, axis=1, keepdims=True)
    S_col = S_i.astype(jnp.float32)

    i16 = lax.broadcasted_iota(jnp.int32, (TD, N_S), 1)
    ohS = (i16 == S_i).astype(jnp.float32)
    ohS1 = (i16 == (S_i - 1)).astype(jnp.float32)
    vS = jnp.dot(ohS, tbl_small, preferred_element_type=jnp.float32,
                 precision=lax.Precision.HIGHEST)
    vS1 = jnp.dot(ohS1, tbl_small[:, 0:1],
                  preferred_element_type=jnp.float32,
                  precision=lax.Precision.HIGHEST)
    vS = jnp.where(S_i < N_S, vS, const_row)
    dS1 = jnp.where((S_i >= 1) & (S_i < N_S + 1), vS1, const_row[:, 0:1])
    dS = vS[:, 0:1]
    oS = vS[:, 1:2]
    oS1 = dS

    ohxT = (lax.broadcasted_iota(jnp.int32, (C, TD), 0) == x_row)
    c_col = lax.broadcasted_iota(jnp.int32, (C, 1), 0).astype(jnp.float32)
    x_col = lax.dot_general(ohxT.astype(jnp.float32), c_col,
                            (((0,), (0,)), ((), ())),
                            preferred_element_type=jnp.float32)
    x_i = x_col.astype(jnp.int32)
    c_iota = lax.broadcasted_iota(jnp.int32, (TD, C), 1)
    bx = c_iota == x_i

    la = jnp.log(dS + EPS)
    lb = jnp.log(oS + EPS)
    logits = jnp.where(bx, la, lb)
    nz = jnp.clip(noise, EPS, 1.0)
    g = -jnp.log(-jnp.log(nz))
    y = logits + g
    m = jnp.max(y, axis=1, keepdims=True)
    x_t = jnp.min(jnp.where(y == m, c_iota, C), axis=1, keepdims=True)
    btm = c_iota == x_t
    OH_t = btm.astype(jnp.float32)

    pred_x0 = jnp.dot(OH_t, W,
                      preferred_element_type=jnp.float32,
                      precision=lax.Precision.HIGHEST) + T_row

    mx = jnp.max(pred_x0, axis=1, keepdims=True)
    sh = pred_x0 - mx
    e = jnp.exp(sh)
    se = jnp.sum(e, axis=1, keepdims=True)
    lse = jnp.log(se)

    lp_tok = jnp.sum(jnp.where(bx, sh, 0.0), axis=1, keepdims=True) - lse
    ce_part = jnp.sum(lp_tok)

    lf1 = jnp.where(btm, lkd, lko)

    denom = 1.0 + (C - 1) * EPS
    f2d = (dS1 + EPS * (1.0 - dS1)) / denom
    f2o = (oS1 + EPS * (1.0 - oS1)) / denom
    true_q = lf1 + jnp.where(bx, jnp.log(f2d + EPS), jnp.log(f2o + EPS))

    fact2_p = oS1 + ((dS1 - oS1) / se) * e
    pred_q = lf1 + jnp.log(fact2_p + EPS)

    mt = jnp.max(true_q, axis=1, keepdims=True)
    sht = true_q - mt
    et = jnp.exp(sht)
    st = jnp.sum(et, axis=1, keepdims=True)
    mp = jnp.max(pred_q, axis=1, keepdims=True)
    shp = pred_q - mp
    ep = jnp.exp(shp)
    sp = jnp.sum(ep, axis=1, keepdims=True)
    dot_ts = jnp.sum(et * (sht - shp), axis=1, keepdims=True)
    kls = dot_ts / st + (jnp.log(sp) - jnp.log(st))

    vb_raw = jnp.sum(kls * S_col)
    return vb_raw, ce_part


def _d3pm_kernel(t_ref, tu_ref, noise_ref, x_ref, beta_ref, tbl_ref, W_ref,
                 T_ref, K_ref, out_ref, acc_ref):
    b = pl.program_id(0)
    tb = t_ref[0, b]

    beta_row = beta_ref[...]
    s1 = lax.broadcasted_iota(jnp.int32, (1, N_T), 1)
    beta_eff = jnp.where(s1 < tb, beta_row, 0.0)
    tbl_small = tbl_ref[0:N_S, 0, 0:2]
    const_row = tbl_ref[N_S:N_S + 1, 0, 0:2]
    W = W_ref[...]
    T_row = T_ref[pl.ds(tb, 1), :]
    lkd = jnp.log(K_ref[0, 0] + EPS)
    lko = jnp.log(K_ref[0, 1] + EPS)

    def _body(h, carry):
        vb_acc, ce_acc = carry
        lo = h * TD
        vb_h, ce_h = _block_sums(tu_ref[pl.ds(lo, TD), :],
                                 noise_ref[pl.ds(lo, TD), :],
                                 x_ref[:, pl.ds(lo, TD)], beta_eff, tbl_small,
                                 const_row, W, T_row, lkd, lko)
        return vb_acc + vb_h, ce_acc + ce_h

    vb_raw, ce_part = lax.fori_loop(0, D // TD, _body, (0.0, 0.0))

    bt_val = jnp.sum(jnp.where(s1 == tb, beta_row, 0.0))
    cum_val = jnp.sum(jnp.where(s1 <= tb, beta_row, 0.0))
    w_b = bt_val / cum_val
    vb_part = vb_raw * (w_b * float(N_T))

    @pl.when(b == 0)
    def _init():
        acc_ref[0, 0] = 0.0
        acc_ref[0, 1] = 0.0

    acc_ref[0, 0] += vb_part
    acc_ref[0, 1] += ce_part

    @pl.when(b == B - 1)
    def _fin():
        n_tok = float(B * D)
        total = 0.001 * (-acc_ref[0, 1] / n_tok) + acc_ref[0, 0] / n_tok
        out_ref[...] = jnp.broadcast_to(total, (1, 1))


def kernel(x, t, trans_u, noise, W, T_emb, K, K_powers, beta_t):
    tu2 = trans_u.reshape(D, B * N_T)
    x3 = x.reshape(B, 1, D).astype(jnp.int32)
    t2 = t.reshape(1, B).astype(jnp.int32)
    beta2 = beta_t.reshape(1, N_T)

    out = pl.pallas_call(
        _d3pm_kernel,
        grid=(B,),
        in_specs=[
            pl.BlockSpec(memory_space=pltpu.SMEM),
            pl.BlockSpec((D, N_T), lambda b: (0, b)),
            pl.BlockSpec((None, D, C), lambda b: (b, 0, 0)),
            pl.BlockSpec((None, 1, D), lambda b: (b, 0, 0)),
            pl.BlockSpec((1, N_T), lambda b: (0, 0)),
            pl.BlockSpec((N_S + 1, 8, C), lambda b: (0, 0, 0)),
            pl.BlockSpec((C, C), lambda b: (0, 0)),
            pl.BlockSpec((N_T, C), lambda b: (0, 0)),
            pl.BlockSpec((C, C), lambda b: (0, 0)),
        ],
        out_specs=pl.BlockSpec((1, 1), lambda b: (0, 0)),
        out_shape=jax.ShapeDtypeStruct((1, 1), jnp.float32),
        scratch_shapes=[pltpu.SMEM((1, 2), jnp.float32)],
    )(t2, tu2, noise, x3, beta2, K_powers, W, T_emb, K)
    return out[0, 0]
